# baseline (speedup 1.0000x reference)
import jax
import jax.numpy as jnp
from jax import lax
from jax.experimental import pallas as pl
from jax.experimental.pallas import tpu as pltpu
from jax.experimental.pallas import tpu_sc as plsc

_NB = 999999
_K = 0x45D9F3B
_R = 16384
_C = 200
_NW = 32
_RW = _R // _NW
_NCH = 4
_CR = _RW // _NCH
_NV = _C // 16
_TAIL = _C - 16


def _lshr(v, k):
    return lax.shift_right_logical(v, jnp.int32(k))


def _hash_mod(v):
    v = v ^ _lshr(v, 16)
    v = v * jnp.int32(_K)
    v = v ^ _lshr(v, 16)
    v = v * jnp.int32(_K)
    v = v ^ _lshr(v, 16)
    q0 = _lshr(_lshr(v, 16) * jnp.int32(4295), 16)
    r = v - q0 * jnp.int32(_NB)
    r = jnp.where(r >= jnp.int32(_NB), r - jnp.int32(_NB), r)
    r = jnp.where(r < 0, r + jnp.int32(_NB), r)
    return r


def _bucketize(v):
    h = _hash_mod(v)
    return jnp.where(v != 0, h + jnp.int32(1), jnp.int32(0))


def _body(x_hbm, o_hbm, ibuf, obuf):
    wid = lax.axis_index("s") * 2 + lax.axis_index("c")
    base = wid * _RW

    def one_chunk(c):
        r0 = base + c * _CR
        pltpu.sync_copy(x_hbm.at[pl.ds(r0, _CR)], ibuf)

        @plsc.parallel_loop(0, _CR, 1, unroll=2)
        def _(r):
            for j in range(_NV):
                v = ibuf[r, pl.ds(j * 16, 16)]
                obuf[r, pl.ds(j * 16, 16)] = _bucketize(v)
            v = ibuf[r, pl.ds(_TAIL, 16)]
            obuf[r, pl.ds(_TAIL, 16)] = _bucketize(v)

        pltpu.sync_copy(obuf, o_hbm.at[pl.ds(r0, _CR)])

    for c in range(_NCH):
        one_chunk(c)


def kernel(x):
    run = pl.kernel(
        _body,
        out_type=jax.ShapeDtypeStruct((_R, _C), jnp.int32),
        mesh=plsc.VectorSubcoreMesh(core_axis_name="c", subcore_axis_name="s"),
        scratch_types=[
            pltpu.VMEM((_CR, _C), jnp.int32),
            pltpu.VMEM((_CR, _C), jnp.int32),
        ],
        compiler_params=pltpu.CompilerParams(use_tc_tiling_on_sc=True),
    )
    return run(x)


# speedup vs baseline: 1.3842x; 1.3842x over previous
import jax
import jax.numpy as jnp
from jax import lax
from jax.experimental import pallas as pl
from jax.experimental.pallas import tpu as pltpu
from jax.experimental.pallas import tpu_sc as plsc

_NB = 999999
_K = 0x45D9F3B
_R = 16384
_C = 200
_NW = 32
_RW = _R // _NW
_NCH = 8
_CR = _RW // _NCH
_NV = _C // 16
_TAIL = _C - 16


def _lshr(v, k):
    return lax.shift_right_logical(v, jnp.int32(k))


def _hash_mod(v):
    v = v ^ _lshr(v, 16)
    v = v * jnp.int32(_K)
    v = v ^ _lshr(v, 16)
    v = v * jnp.int32(_K)
    s = _lshr(v, 16)
    w = v ^ s
    q0 = _lshr(s * jnp.int32(8589), 17)
    r = w - q0 * jnp.int32(_NB)
    r = jnp.where(r >= jnp.int32(_NB), r - jnp.int32(_NB), r)
    return r


def _bucketize(v):
    h = _hash_mod(v)
    return jnp.where(v != 0, h + jnp.int32(1), jnp.int32(0))


def _body(x_hbm, o_hbm, ibuf, obuf, isem, osem):
    wid = lax.axis_index("s") * 2 + lax.axis_index("c")
    base = wid * _RW

    def in_copy(c, p):
        return pltpu.make_async_copy(
            x_hbm.at[pl.ds(base + c * _CR, _CR)], ibuf.at[p], isem.at[p]
        )

    def out_copy(c, p):
        return pltpu.make_async_copy(
            obuf.at[p], o_hbm.at[pl.ds(base + c * _CR, _CR)], osem.at[p]
        )

    in_copy(0, 0).start()

    def one_chunk(c, carry):
        p = lax.rem(c, 2)
        in_copy(c, p).wait()

        @pl.when(c + 1 < _NCH)
        def _():
            in_copy(c + 1, 1 - p).start()

        @pl.when(c >= 2)
        def _():
            out_copy(c - 2, p).wait()

        @plsc.parallel_loop(0, _CR, 1)
        def _(r):
            for j in range(_NV):
                v = ibuf[p, r, pl.ds(j * 16, 16)]
                obuf[p, r, pl.ds(j * 16, 16)] = _bucketize(v)
            v = ibuf[p, r, pl.ds(_TAIL, 16)]
            obuf[p, r, pl.ds(_TAIL, 16)] = _bucketize(v)

        out_copy(c, p).start()
        return carry

    lax.fori_loop(0, _NCH, one_chunk, 0)
    out_copy(_NCH - 2, 0).wait()
    out_copy(_NCH - 1, 1).wait()


def kernel(x):
    run = pl.kernel(
        _body,
        out_type=jax.ShapeDtypeStruct((_R, _C), jnp.int32),
        mesh=plsc.VectorSubcoreMesh(core_axis_name="c", subcore_axis_name="s"),
        scratch_types=[
            pltpu.VMEM((2, _CR, _C), jnp.int32),
            pltpu.VMEM((2, _CR, _C), jnp.int32),
            pltpu.SemaphoreType.DMA((2,)),
            pltpu.SemaphoreType.DMA((2,)),
        ],
        compiler_params=pltpu.CompilerParams(use_tc_tiling_on_sc=True),
    )
    return run(x)
